# baseline (speedup 1.0000x reference)
import dataclasses
import functools

import jax
import jax.numpy as jnp
from jax import lax
from jax.experimental import pallas as pl
from jax.experimental.pallas import tpu as pltpu
from jax.experimental.pallas import tpu_sc as plsc

N = 10000
E = 160000
D = 128
DP = 64
DF = 192
N2 = 10240
E2 = 163840

NC = 2
NS = 16
NW = NC * NS

EPW = E // NW
GCH = 200
GN = EPW // GCH

EPW2 = E2 // NW
PCH = 1280
PN = EPW2 // PCH

EPS = E // NS
SCH = 160
SNF = EPS // SCH
STL = EPS - SNF * SCH

ZPS = N2 // NS

_sc_mesh = lambda: plsc.VectorSubcoreMesh(core_axis_name="c", subcore_axis_name="s")


def _no_layout_cp():
    cp = pltpu.CompilerParams()
    if "needs_layout_passes" in pltpu.CompilerParams.__dataclass_fields__:
        cp = dataclasses.replace(cp, needs_layout_passes=False)
    return cp


def _sc_gather_rows(table, idx):
    @functools.partial(
        pl.kernel,
        mesh=_sc_mesh(),
        out_type=jax.ShapeDtypeStruct((E, D), jnp.float32),
        scratch_types=[
            pltpu.VMEM((EPW,), jnp.int32),
            pltpu.VMEM((GCH, D), jnp.float32),
            pltpu.VMEM((GCH, D), jnp.float32),
            pltpu.SemaphoreType.DMA,
            pltpu.SemaphoreType.DMA,
            pltpu.SemaphoreType.DMA,
        ],
    )
    def k(table_hbm, idx_hbm, out_hbm, idx_v, rows0, rows1, sg, sw0, sw1):
        wid = lax.axis_index("s") * NC + lax.axis_index("c")
        base = wid * EPW
        pltpu.sync_copy(idx_hbm.at[pl.ds(base, EPW)], idx_v)
        rows = (rows0, rows1)
        sw = (sw0, sw1)

        def gather_sync(ci, b):
            pltpu.async_copy(
                table_hbm.at[idx_v.at[pl.ds(ci * GCH, GCH)]], rows[b], sg
            ).wait()

        def write_start(ci, b):
            pltpu.async_copy(rows[b], out_hbm.at[pl.ds(base + ci * GCH, GCH)], sw[b])

        def write_wait(b):
            pltpu.make_async_copy(rows[b], out_hbm.at[pl.ds(base, GCH)], sw[b]).wait()

        @pl.loop(0, (GN - 1) // 2)
        def _(ii):
            c0 = ii * 2

            @pl.when(ii > 0)
            def _():
                write_wait(0)

            gather_sync(c0, 0)
            write_start(c0, 0)

            @pl.when(ii > 0)
            def _():
                write_wait(1)

            gather_sync(c0 + 1, 1)
            write_start(c0 + 1, 1)

        write_wait(0)
        gather_sync(GN - 1, 0)
        write_start(GN - 1, 0)
        write_wait(0)
        write_wait(1)

    return k(table, idx)


def _sc_gather_pos(pos_flat, srcp, dstp):
    @functools.partial(
        pl.kernel,
        mesh=_sc_mesh(),
        out_type=[
            jax.ShapeDtypeStruct((8, E2), jnp.float32),
            jax.ShapeDtypeStruct((8, E2), jnp.float32),
        ],
        compiler_params=_no_layout_cp(),
        scratch_types=[
            pltpu.VMEM((N2 * 4,), jnp.float32),
            pltpu.VMEM((PCH,), jnp.int32),
            pltpu.VMEM((PCH,), jnp.int32),
            pltpu.VMEM((8, PCH), jnp.float32),
            pltpu.VMEM((8, PCH), jnp.float32),
            pltpu.SemaphoreType.DMA,
        ],
    )
    def k(pos_hbm, src_hbm, dst_hbm, pit_hbm, pjt_hbm,
          pos_v, sidx, didx, pit_v, pjt_v, sem):
        wid = lax.axis_index("s") * NC + lax.axis_index("c")
        base = wid * EPW2
        pltpu.sync_copy(pos_hbm, pos_v)

        @pl.loop(0, PCH // 16)
        def _(i):
            z = jnp.zeros((16,), jnp.float32)
            for r in range(4, 8):
                pit_v[r, pl.ds(i * 16, 16)] = z
            for r in range(3, 8):
                pjt_v[r, pl.ds(i * 16, 16)] = z

        @pl.loop(0, PN)
        def _(ci):
            off = base + ci * PCH
            pltpu.sync_copy(src_hbm.at[pl.ds(off, PCH)], sidx)
            pltpu.sync_copy(dst_hbm.at[pl.ds(off, PCH)], didx)

            @pl.loop(0, PCH // 16)
            def _(kk):
                sl = pl.ds(kk * 16, 16)
                s16 = sidx[sl] * 4
                d16 = didx[sl] * 4
                pcoord = []
                for col in range(3):
                    pj_c = plsc.load_gather(pos_v, [s16 + col])
                    pi_c = plsc.load_gather(pos_v, [d16 + col])
                    pjt_v[col, sl] = pj_c
                    pit_v[col, sl] = pi_c
                    pcoord.append((pi_c, pj_c))
                dx = pcoord[0][0] - pcoord[0][1]
                dy = pcoord[1][0] - pcoord[1][1]
                dz = pcoord[2][0] - pcoord[2][1]
                pit_v[3, sl] = dx * dx + dy * dy + dz * dz

            pltpu.sync_copy(pit_v, pit_hbm.at[:, pl.ds(off, PCH)])
            pltpu.sync_copy(pjt_v, pjt_hbm.at[:, pl.ds(off, PCH)])

    return k(pos_flat, srcp, dstp)


def _sc_scatter_add2(msgA, msgB, dst, zeros):
    @functools.partial(
        pl.kernel,
        mesh=_sc_mesh(),
        out_type=[
            jax.ShapeDtypeStruct((N2, D), jnp.float32),
            jax.ShapeDtypeStruct((N2, D), jnp.float32),
        ],
        scratch_types=[
            pltpu.VMEM((SCH,), jnp.int32),
            pltpu.VMEM((SCH,), jnp.int32),
            pltpu.VMEM((STL,), jnp.int32),
            pltpu.VMEM((SCH, D), jnp.float32),
            pltpu.VMEM((SCH, D), jnp.float32),
            pltpu.VMEM_SHARED((N2, D), jnp.float32),
            pltpu.SemaphoreType.DMA,
            pltpu.SemaphoreType.DMA,
        ],
    )
    def k(msgA_hbm, msgB_hbm, dst_hbm, z_hbm, outA_hbm, outB_hbm,
          idx0, idx1, idx_t, rows0, rows1, acc_sh, sl0, sl1):
        c = lax.axis_index("c")
        s = lax.axis_index("s")
        pltpu.sync_copy(z_hbm.at[pl.ds(s * ZPS, ZPS)], acc_sh.at[pl.ds(s * ZPS, ZPS)])
        plsc.subcore_barrier()
        base = s * EPS
        idxs = (idx0, idx1)
        rows = (rows0, rows1)
        sl = (sl0, sl1)

        def run_pass(msg_hbm, out_hbm):
            def load_start(ci, b):
                pltpu.async_copy(dst_hbm.at[pl.ds(base + ci * SCH, SCH)], idxs[b], sl[b])
                pltpu.async_copy(msg_hbm.at[pl.ds(base + ci * SCH, SCH)], rows[b], sl[b])

            def load_wait(b):
                pltpu.make_async_copy(dst_hbm.at[pl.ds(base, SCH)], idxs[b], sl[b]).wait()
                pltpu.make_async_copy(msg_hbm.at[pl.ds(base, SCH)], rows[b], sl[b]).wait()

            def scat_sync(b):
                pltpu.sync_copy(rows[b], acc_sh.at[idxs[b]], add=True)

            load_start(0, 0)
            load_start(1, 1)

            @pl.loop(0, (SNF - 2) // 2)
            def _(ii):
                c0 = ii * 2
                load_wait(0)
                scat_sync(0)
                load_start(c0 + 2, 0)
                load_wait(1)
                scat_sync(1)
                load_start(c0 + 3, 1)

            load_wait(0)
            scat_sync(0)
            load_wait(1)
            scat_sync(1)

            toff = base + SNF * SCH
            pltpu.sync_copy(dst_hbm.at[pl.ds(toff, STL)], idx_t)
            pltpu.sync_copy(msg_hbm.at[pl.ds(toff, STL)], rows0.at[pl.ds(0, STL)])
            pltpu.sync_copy(rows0.at[pl.ds(0, STL)], acc_sh.at[idx_t], add=True)

            plsc.subcore_barrier()
            pltpu.sync_copy(acc_sh.at[pl.ds(s * ZPS, ZPS)], out_hbm.at[pl.ds(s * ZPS, ZPS)])

        @pl.when(c == 0)
        def _():
            run_pass(msgA_hbm, outA_hbm)

        @pl.when(c == 1)
        def _():
            run_pass(msgB_hbm, outB_hbm)

    return k(msgA, msgB, dst, zeros)


BE = 1280


def _edge_body(xj_ref, pit_ref, pjt_ref, w8_ref, ppb_ref, aW_ref, ab_ref,
               msgA_ref, msgB_ref):
    pit = pit_ref[...]
    pjt = pjt_ref[...]
    dij = jnp.sqrt(pit[3:4, :])
    m8 = jnp.concatenate(
        [pit[0:3, :], pjt[0:3, :], dij, jnp.zeros((1, BE), jnp.float32)], axis=0)
    rij = lax.dot_general(
        m8, w8_ref[...], (((0,), (0,)), ((), ())),
        preferred_element_type=jnp.float32) + ppb_ref[...]
    rij = jnp.maximum(rij, 0.0)
    fij = jnp.concatenate([xj_ref[...], rij], axis=1)
    g = jnp.dot(fij.astype(jnp.bfloat16), aW_ref[...],
                preferred_element_type=jnp.float32) + ab_ref[...]
    g = jnp.maximum(g, 0.0)
    m = jnp.max(g, axis=1, keepdims=True)
    ex = jnp.exp(g - m)
    sm = ex / jnp.sum(ex, axis=1, keepdims=True)
    smf = sm * fij
    msgA_ref[...] = smf[:, :D]
    msgB_ref[...] = jnp.concatenate(
        [smf[:, D:], jnp.zeros((BE, D - DP), jnp.float32)], axis=1)


def _tc_edge(xj, pit, pjt, w8, ppb, aW, ab):
    return pl.pallas_call(
        _edge_body,
        grid=(E // BE,),
        in_specs=[
            pl.BlockSpec((BE, D), lambda i: (i, 0)),
            pl.BlockSpec((8, BE), lambda i: (0, i)),
            pl.BlockSpec((8, BE), lambda i: (0, i)),
            pl.BlockSpec((8, DP), lambda i: (0, 0)),
            pl.BlockSpec((1, DP), lambda i: (0, 0)),
            pl.BlockSpec((DF, DF), lambda i: (0, 0)),
            pl.BlockSpec((1, DF), lambda i: (0, 0)),
        ],
        out_specs=[
            pl.BlockSpec((BE, D), lambda i: (i, 0)),
            pl.BlockSpec((BE, D), lambda i: (i, 0)),
        ],
        out_shape=[
            jax.ShapeDtypeStruct((E, D), jnp.float32),
            jax.ShapeDtypeStruct((E, D), jnp.float32),
        ],
    )(xj, pit, pjt, w8, ppb, aW, ab)


BN = 1024


def _update_body(a_ref, b_ref, gW_ref, gb_ref, o_ref):
    aggr = jnp.concatenate([a_ref[...], b_ref[:, :DP]], axis=1)
    h = jnp.dot(aggr, gW_ref[...], preferred_element_type=jnp.float32) + gb_ref[...]
    o_ref[...] = jnp.maximum(h, 0.0)


def _tc_update(pA, pB, gW, gb):
    return pl.pallas_call(
        _update_body,
        grid=(N2 // BN,),
        in_specs=[
            pl.BlockSpec((BN, D), lambda i: (i, 0)),
            pl.BlockSpec((BN, D), lambda i: (i, 0)),
            pl.BlockSpec((DF, D), lambda i: (0, 0)),
            pl.BlockSpec((1, D), lambda i: (0, 0)),
        ],
        out_specs=pl.BlockSpec((BN, D), lambda i: (i, 0)),
        out_shape=jax.ShapeDtypeStruct((N2, D), jnp.float32),
    )(pA, pB, gW, gb)


def _final_body(a_ref, b_ref, x_ref, gW_ref, gb_ref, scW_ref, scb_ref, o_ref):
    aggr = jnp.concatenate([a_ref[...], b_ref[:, :DP]], axis=1)
    h = jnp.dot(aggr, gW_ref[...], preferred_element_type=jnp.float32) + gb_ref[...]
    h = jnp.maximum(h, 0.0)
    sc = jnp.dot(x_ref[...], scW_ref[...], preferred_element_type=jnp.float32) + scb_ref[...]
    o_ref[...] = jnp.maximum(h + sc, 0.0)


def _tc_final(pA, pB, x, gW, gb, scW, scb):
    return pl.pallas_call(
        _final_body,
        grid=(N2 // BN,),
        in_specs=[
            pl.BlockSpec((BN, D), lambda i: (i, 0)),
            pl.BlockSpec((BN, D), lambda i: (i, 0)),
            pl.BlockSpec((BN, D), lambda i: (i, 0)),
            pl.BlockSpec((DF, D), lambda i: (0, 0)),
            pl.BlockSpec((1, D), lambda i: (0, 0)),
            pl.BlockSpec((D, D), lambda i: (0, 0)),
            pl.BlockSpec((1, D), lambda i: (0, 0)),
        ],
        out_specs=pl.BlockSpec((BN, D), lambda i: (i, 0)),
        out_shape=jax.ShapeDtypeStruct((N2, D), jnp.float32),
    )(pA, pB, x, gW, gb, scW, scb)


def _prep_pp(ppW):
    return jnp.concatenate([
        ppW[0:3] + ppW[6:9],
        ppW[3:6] - ppW[6:9],
        ppW[9:10],
        jnp.zeros((1, DP), jnp.float32),
    ], axis=0)


def kernel(x, pos, edge_index, ppW1, ppb1, aW1, ab1, gW1, gb1,
           ppW2, ppb2, aW2, ab2, gW2, gb2, scW, scb):
    src = edge_index[0]
    dst = edge_index[1]
    srcp = jnp.zeros((E2,), jnp.int32).at[:E].set(src)
    dstp = jnp.zeros((E2,), jnp.int32).at[:E].set(dst)

    xp = jnp.zeros((N2, D), jnp.float32).at[:N].set(x)
    pos4 = jnp.zeros((N2, 4), jnp.float32).at[:N, :3].set(pos)
    zeros = jnp.zeros((N2, D), jnp.float32)

    w81 = _prep_pp(ppW1)
    w82 = _prep_pp(ppW2)
    ppb1r = ppb1.reshape(1, DP)
    ppb2r = ppb2.reshape(1, DP)
    ab1r = ab1.reshape(1, DF)
    ab2r = ab2.reshape(1, DF)
    gb1r = gb1.reshape(1, D)
    gb2r = gb2.reshape(1, D)
    scbr = scb.reshape(1, D)

    pit, pjt = _sc_gather_pos(pos4.reshape(-1), srcp, dstp)

    xj1 = _sc_gather_rows(xp, src)
    msgA1, msgB1 = _tc_edge(xj1, pit, pjt, w81, ppb1r, aW1.astype(jnp.bfloat16), ab1r)
    pA1, pB1 = _sc_scatter_add2(msgA1, msgB1, dst, zeros)
    h1 = _tc_update(pA1, pB1, gW1, gb1r)

    xj2 = _sc_gather_rows(h1, src)
    msgA2, msgB2 = _tc_edge(xj2, pit, pjt, w82, ppb2r, aW2.astype(jnp.bfloat16), ab2r)
    pA2, pB2 = _sc_scatter_add2(msgA2, msgB2, dst, zeros)
    out = _tc_final(pA2, pB2, xp, gW2, gb2r, scW, scbr)

    return out[:N]


# speedup vs baseline: 1.0058x; 1.0058x over previous
import dataclasses
import functools

import jax
import jax.numpy as jnp
from jax import lax
from jax.experimental import pallas as pl
from jax.experimental.pallas import tpu as pltpu
from jax.experimental.pallas import tpu_sc as plsc

N = 10000
E = 160000
D = 128
DP = 64
DF = 192
N2 = 10240
E2 = 163840

NC = 2
NS = 16
NW = NC * NS

EPW = E // NW
GCH = 384
GN = EPW // GCH
GTL = EPW - GN * GCH

EPW2 = E2 // NW
PCH = 1280
PN = EPW2 // PCH

EPS = E // NS
SCH = 160
SNF = EPS // SCH
STL = EPS - SNF * SCH

ZPS = N2 // NS

_sc_mesh = lambda: plsc.VectorSubcoreMesh(core_axis_name="c", subcore_axis_name="s")


def _no_layout_cp():
    cp = pltpu.CompilerParams()
    if "needs_layout_passes" in pltpu.CompilerParams.__dataclass_fields__:
        cp = dataclasses.replace(cp, needs_layout_passes=False)
    return cp


def _sc_gather_rows(table, idx):
    @functools.partial(
        pl.kernel,
        mesh=_sc_mesh(),
        out_type=jax.ShapeDtypeStruct((E, D), jnp.float32),
        scratch_types=[
            pltpu.VMEM((EPW,), jnp.int32),
            pltpu.VMEM((GCH, D), jnp.float32),
            pltpu.VMEM((GCH, D), jnp.float32),
            pltpu.SemaphoreType.DMA,
            pltpu.SemaphoreType.DMA,
            pltpu.SemaphoreType.DMA,
        ],
    )
    def k(table_hbm, idx_hbm, out_hbm, idx_v, rows0, rows1, sg, sw0, sw1):
        wid = lax.axis_index("s") * NC + lax.axis_index("c")
        base = wid * EPW
        pltpu.sync_copy(idx_hbm.at[pl.ds(base, EPW)], idx_v)
        rows = (rows0, rows1)
        sw = (sw0, sw1)

        def gather_sync(ci, b):
            pltpu.async_copy(
                table_hbm.at[idx_v.at[pl.ds(ci * GCH, GCH)]], rows[b], sg
            ).wait()

        def write_start(ci, b):
            pltpu.async_copy(rows[b], out_hbm.at[pl.ds(base + ci * GCH, GCH)], sw[b])

        def write_wait(b):
            pltpu.make_async_copy(rows[b], out_hbm.at[pl.ds(base, GCH)], sw[b]).wait()

        def gather_sync_tail(ci):
            pltpu.async_copy(
                table_hbm.at[idx_v.at[pl.ds(ci * GCH, GTL)]],
                rows0.at[pl.ds(0, GTL)], sg,
            ).wait()

        @pl.loop(0, (GN - 1) // 2)
        def _(ii):
            c0 = ii * 2

            @pl.when(ii > 0)
            def _():
                write_wait(0)

            gather_sync(c0, 0)
            write_start(c0, 0)

            @pl.when(ii > 0)
            def _():
                write_wait(1)

            gather_sync(c0 + 1, 1)
            write_start(c0 + 1, 1)

        write_wait(0)
        gather_sync(GN - 1, 0)
        write_start(GN - 1, 0)
        write_wait(0)
        write_wait(1)

        gather_sync_tail(GN)
        pltpu.sync_copy(rows0.at[pl.ds(0, GTL)],
                        out_hbm.at[pl.ds(base + GN * GCH, GTL)])

    return k(table, idx)


def _sc_gather_pos(pos_flat, srcp, dstp):
    @functools.partial(
        pl.kernel,
        mesh=_sc_mesh(),
        out_type=[
            jax.ShapeDtypeStruct((8, E2), jnp.float32),
            jax.ShapeDtypeStruct((8, E2), jnp.float32),
        ],
        compiler_params=_no_layout_cp(),
        scratch_types=[
            pltpu.VMEM((N2 * 4,), jnp.float32),
            pltpu.VMEM((PCH,), jnp.int32),
            pltpu.VMEM((PCH,), jnp.int32),
            pltpu.VMEM((8, PCH), jnp.float32),
            pltpu.VMEM((8, PCH), jnp.float32),
            pltpu.SemaphoreType.DMA,
        ],
    )
    def k(pos_hbm, src_hbm, dst_hbm, pit_hbm, pjt_hbm,
          pos_v, sidx, didx, pit_v, pjt_v, sem):
        wid = lax.axis_index("s") * NC + lax.axis_index("c")
        base = wid * EPW2
        pltpu.sync_copy(pos_hbm, pos_v)

        @pl.loop(0, PCH // 16)
        def _(i):
            z = jnp.zeros((16,), jnp.float32)
            for r in range(4, 8):
                pit_v[r, pl.ds(i * 16, 16)] = z
            for r in range(3, 8):
                pjt_v[r, pl.ds(i * 16, 16)] = z

        @pl.loop(0, PN)
        def _(ci):
            off = base + ci * PCH
            pltpu.sync_copy(src_hbm.at[pl.ds(off, PCH)], sidx)
            pltpu.sync_copy(dst_hbm.at[pl.ds(off, PCH)], didx)

            @pl.loop(0, PCH // 16)
            def _(kk):
                sl = pl.ds(kk * 16, 16)
                s16 = sidx[sl] * 4
                d16 = didx[sl] * 4
                pcoord = []
                for col in range(3):
                    pj_c = plsc.load_gather(pos_v, [s16 + col])
                    pi_c = plsc.load_gather(pos_v, [d16 + col])
                    pjt_v[col, sl] = pj_c
                    pit_v[col, sl] = pi_c
                    pcoord.append((pi_c, pj_c))
                dx = pcoord[0][0] - pcoord[0][1]
                dy = pcoord[1][0] - pcoord[1][1]
                dz = pcoord[2][0] - pcoord[2][1]
                pit_v[3, sl] = dx * dx + dy * dy + dz * dz

            pltpu.sync_copy(pit_v, pit_hbm.at[:, pl.ds(off, PCH)])
            pltpu.sync_copy(pjt_v, pjt_hbm.at[:, pl.ds(off, PCH)])

    return k(pos_flat, srcp, dstp)


def _sc_scatter_add2(msgA, msgB, dst, zeros):
    @functools.partial(
        pl.kernel,
        mesh=_sc_mesh(),
        out_type=[
            jax.ShapeDtypeStruct((N2, D), jnp.float32),
            jax.ShapeDtypeStruct((N2, D), jnp.float32),
        ],
        scratch_types=[
            pltpu.VMEM((SCH,), jnp.int32),
            pltpu.VMEM((SCH,), jnp.int32),
            pltpu.VMEM((STL,), jnp.int32),
            pltpu.VMEM((SCH, D), jnp.float32),
            pltpu.VMEM((SCH, D), jnp.float32),
            pltpu.VMEM_SHARED((N2, D), jnp.float32),
            pltpu.SemaphoreType.DMA,
            pltpu.SemaphoreType.DMA,
        ],
    )
    def k(msgA_hbm, msgB_hbm, dst_hbm, z_hbm, outA_hbm, outB_hbm,
          idx0, idx1, idx_t, rows0, rows1, acc_sh, sl0, sl1):
        c = lax.axis_index("c")
        s = lax.axis_index("s")
        pltpu.sync_copy(z_hbm.at[pl.ds(s * ZPS, ZPS)], acc_sh.at[pl.ds(s * ZPS, ZPS)])
        plsc.subcore_barrier()
        base = s * EPS
        idxs = (idx0, idx1)
        rows = (rows0, rows1)
        sl = (sl0, sl1)

        def run_pass(msg_hbm, out_hbm):
            def load_start(ci, b):
                pltpu.async_copy(dst_hbm.at[pl.ds(base + ci * SCH, SCH)], idxs[b], sl[b])
                pltpu.async_copy(msg_hbm.at[pl.ds(base + ci * SCH, SCH)], rows[b], sl[b])

            def load_wait(b):
                pltpu.make_async_copy(dst_hbm.at[pl.ds(base, SCH)], idxs[b], sl[b]).wait()
                pltpu.make_async_copy(msg_hbm.at[pl.ds(base, SCH)], rows[b], sl[b]).wait()

            def scat_sync(b):
                pltpu.sync_copy(rows[b], acc_sh.at[idxs[b]], add=True)

            load_start(0, 0)
            load_start(1, 1)

            @pl.loop(0, (SNF - 2) // 2)
            def _(ii):
                c0 = ii * 2
                load_wait(0)
                scat_sync(0)
                load_start(c0 + 2, 0)
                load_wait(1)
                scat_sync(1)
                load_start(c0 + 3, 1)

            load_wait(0)
            scat_sync(0)
            load_wait(1)
            scat_sync(1)

            toff = base + SNF * SCH
            pltpu.sync_copy(dst_hbm.at[pl.ds(toff, STL)], idx_t)
            pltpu.sync_copy(msg_hbm.at[pl.ds(toff, STL)], rows0.at[pl.ds(0, STL)])
            pltpu.sync_copy(rows0.at[pl.ds(0, STL)], acc_sh.at[idx_t], add=True)

            plsc.subcore_barrier()
            pltpu.sync_copy(acc_sh.at[pl.ds(s * ZPS, ZPS)], out_hbm.at[pl.ds(s * ZPS, ZPS)])

        @pl.when(c == 0)
        def _():
            run_pass(msgA_hbm, outA_hbm)

        @pl.when(c == 1)
        def _():
            run_pass(msgB_hbm, outB_hbm)

    return k(msgA, msgB, dst, zeros)


BE = 1280


def _edge_body(xj_ref, pit_ref, pjt_ref, w8_ref, ppb_ref, aW_ref, ab_ref,
               msgA_ref, msgB_ref):
    pit = pit_ref[...]
    pjt = pjt_ref[...]
    dij = jnp.sqrt(pit[3:4, :])
    m8 = jnp.concatenate(
        [pit[0:3, :], pjt[0:3, :], dij, jnp.zeros((1, BE), jnp.float32)], axis=0)
    rij = lax.dot_general(
        m8, w8_ref[...], (((0,), (0,)), ((), ())),
        preferred_element_type=jnp.float32) + ppb_ref[...]
    rij = jnp.maximum(rij, 0.0)
    fij = jnp.concatenate([xj_ref[...], rij], axis=1)
    g = jnp.dot(fij, aW_ref[...], preferred_element_type=jnp.float32) + ab_ref[...]
    g = jnp.maximum(g, 0.0)
    m = jnp.max(g, axis=1, keepdims=True)
    ex = jnp.exp(g - m)
    sm = ex / jnp.sum(ex, axis=1, keepdims=True)
    smf = sm * fij
    msgA_ref[...] = smf[:, :D]
    msgB_ref[...] = jnp.concatenate(
        [smf[:, D:], jnp.zeros((BE, D - DP), jnp.float32)], axis=1)


def _tc_edge(xj, pit, pjt, w8, ppb, aW, ab):
    return pl.pallas_call(
        _edge_body,
        grid=(E // BE,),
        in_specs=[
            pl.BlockSpec((BE, D), lambda i: (i, 0)),
            pl.BlockSpec((8, BE), lambda i: (0, i)),
            pl.BlockSpec((8, BE), lambda i: (0, i)),
            pl.BlockSpec((8, DP), lambda i: (0, 0)),
            pl.BlockSpec((1, DP), lambda i: (0, 0)),
            pl.BlockSpec((DF, DF), lambda i: (0, 0)),
            pl.BlockSpec((1, DF), lambda i: (0, 0)),
        ],
        out_specs=[
            pl.BlockSpec((BE, D), lambda i: (i, 0)),
            pl.BlockSpec((BE, D), lambda i: (i, 0)),
        ],
        out_shape=[
            jax.ShapeDtypeStruct((E, D), jnp.float32),
            jax.ShapeDtypeStruct((E, D), jnp.float32),
        ],
    )(xj, pit, pjt, w8, ppb, aW, ab)


BN = 1024


def _update_body(a_ref, b_ref, gW_ref, gb_ref, o_ref):
    aggr = jnp.concatenate([a_ref[...], b_ref[:, :DP]], axis=1)
    h = jnp.dot(aggr, gW_ref[...], preferred_element_type=jnp.float32) + gb_ref[...]
    o_ref[...] = jnp.maximum(h, 0.0)


def _tc_update(pA, pB, gW, gb):
    return pl.pallas_call(
        _update_body,
        grid=(N2 // BN,),
        in_specs=[
            pl.BlockSpec((BN, D), lambda i: (i, 0)),
            pl.BlockSpec((BN, D), lambda i: (i, 0)),
            pl.BlockSpec((DF, D), lambda i: (0, 0)),
            pl.BlockSpec((1, D), lambda i: (0, 0)),
        ],
        out_specs=pl.BlockSpec((BN, D), lambda i: (i, 0)),
        out_shape=jax.ShapeDtypeStruct((N2, D), jnp.float32),
    )(pA, pB, gW, gb)


def _final_body(a_ref, b_ref, x_ref, gW_ref, gb_ref, scW_ref, scb_ref, o_ref):
    aggr = jnp.concatenate([a_ref[...], b_ref[:, :DP]], axis=1)
    h = jnp.dot(aggr, gW_ref[...], preferred_element_type=jnp.float32) + gb_ref[...]
    h = jnp.maximum(h, 0.0)
    sc = jnp.dot(x_ref[...], scW_ref[...], preferred_element_type=jnp.float32) + scb_ref[...]
    o_ref[...] = jnp.maximum(h + sc, 0.0)


def _tc_final(pA, pB, x, gW, gb, scW, scb):
    return pl.pallas_call(
        _final_body,
        grid=(N2 // BN,),
        in_specs=[
            pl.BlockSpec((BN, D), lambda i: (i, 0)),
            pl.BlockSpec((BN, D), lambda i: (i, 0)),
            pl.BlockSpec((BN, D), lambda i: (i, 0)),
            pl.BlockSpec((DF, D), lambda i: (0, 0)),
            pl.BlockSpec((1, D), lambda i: (0, 0)),
            pl.BlockSpec((D, D), lambda i: (0, 0)),
            pl.BlockSpec((1, D), lambda i: (0, 0)),
        ],
        out_specs=pl.BlockSpec((BN, D), lambda i: (i, 0)),
        out_shape=jax.ShapeDtypeStruct((N2, D), jnp.float32),
    )(pA, pB, x, gW, gb, scW, scb)


def _prep_pp(ppW):
    return jnp.concatenate([
        ppW[0:3] + ppW[6:9],
        ppW[3:6] - ppW[6:9],
        ppW[9:10],
        jnp.zeros((1, DP), jnp.float32),
    ], axis=0)


def kernel(x, pos, edge_index, ppW1, ppb1, aW1, ab1, gW1, gb1,
           ppW2, ppb2, aW2, ab2, gW2, gb2, scW, scb):
    src = edge_index[0]
    dst = edge_index[1]
    srcp = jnp.zeros((E2,), jnp.int32).at[:E].set(src)
    dstp = jnp.zeros((E2,), jnp.int32).at[:E].set(dst)

    xp = jnp.zeros((N2, D), jnp.float32).at[:N].set(x)
    pos4 = jnp.zeros((N2, 4), jnp.float32).at[:N, :3].set(pos)
    zeros = jnp.zeros((N2, D), jnp.float32)

    w81 = _prep_pp(ppW1)
    w82 = _prep_pp(ppW2)
    ppb1r = ppb1.reshape(1, DP)
    ppb2r = ppb2.reshape(1, DP)
    ab1r = ab1.reshape(1, DF)
    ab2r = ab2.reshape(1, DF)
    gb1r = gb1.reshape(1, D)
    gb2r = gb2.reshape(1, D)
    scbr = scb.reshape(1, D)

    pit, pjt = _sc_gather_pos(pos4.reshape(-1), srcp, dstp)

    xj1 = _sc_gather_rows(xp, src)
    msgA1, msgB1 = _tc_edge(xj1, pit, pjt, w81, ppb1r, aW1, ab1r)
    pA1, pB1 = _sc_scatter_add2(msgA1, msgB1, dst, zeros)
    h1 = _tc_update(pA1, pB1, gW1, gb1r)

    xj2 = _sc_gather_rows(h1, src)
    msgA2, msgB2 = _tc_edge(xj2, pit, pjt, w82, ppb2r, aW2, ab2r)
    pA2, pB2 = _sc_scatter_add2(msgA2, msgB2, dst, zeros)
    out = _tc_final(pA2, pB2, xp, gW2, gb2r, scW, scbr)

    return out[:N]
